# baseline (speedup 1.0000x reference)
import functools

import jax
import jax.numpy as jnp
import numpy as np
from jax.experimental import pallas as pl

DIM = 256
N_Q = 8
BINS = 1024
HOP = 320
TBLK = 512


def _rvq_kernel(x_ref, cb_ref, ehi_ref, emid_ref, elo_ref, esq_ref,
                quant_ref, codes_ref, loss_ref, subq_ref, *, t_total):
    t = pl.program_id(1)
    tb = x_ref.shape[2]
    hb = tb // 2
    iota = jax.lax.broadcasted_iota(jnp.int32, (BINS, hb), 0)
    lane = jax.lax.broadcasted_iota(jnp.int32, (1, hb), 1)
    dn = (((1,), (0,)), ((), ()))
    rs = [x_ref[0, :, pl.ds(h * hb, hb)] for h in range(2)]
    qsums = [jnp.zeros_like(rs[0]) for _ in range(2)]
    valids = [(t * tb + h * hb + lane) < t_total for h in range(2)]
    for q in range(N_Q):
        e = cb_ref[q]
        lrows = []
        for h in range(2):
            r = rs[h]
            eg = jax.lax.dot_general(
                e, r, dn, preferred_element_type=jnp.float32)
            rsq = jnp.sum(r * r, axis=0, keepdims=True)
            dist = (rsq - 2.0 * eg) + esq_ref[q][:, None]
            codes = jnp.argmin(dist, axis=0).astype(jnp.int32)
            onehot = (iota == codes[None, :]).astype(jnp.bfloat16)
            v = ((jax.lax.dot_general(ehi_ref[q], onehot, dn,
                                      preferred_element_type=jnp.float32)
                  + jax.lax.dot_general(emid_ref[q], onehot, dn,
                                        preferred_element_type=jnp.float32))
                 + jax.lax.dot_general(elo_ref[q], onehot, dn,
                                       preferred_element_type=jnp.float32))
            d = v - r
            qsums[h] = qsums[h] + (r + d)
            dm = jnp.where(valids[h], d, 0.0)
            lrows.append(jnp.full((128,), jnp.sum(dm * dm), jnp.float32))
            codes_ref[0, q, pl.ds(h * hb, hb)] = codes
            subq_ref[q, 0, :, pl.ds(h * hb, hb)] = v
            rs[h] = r - v
        lrow = lrows[0] + lrows[1]
        loss_ref[0, q, :] = jnp.where(t == 0, lrow, loss_ref[0, q, :] + lrow)
    quant_ref[0, :, pl.ds(0, hb)] = qsums[0]
    quant_ref[0, :, pl.ds(hb, hb)] = qsums[1]


def _plane_kernel(cbt_ref, hi_ref, mid_ref, lo_ref):
    e = cbt_ref[...]
    ehi = e.astype(jnp.bfloat16)
    r1 = e - ehi.astype(jnp.float32)
    emid = r1.astype(jnp.bfloat16)
    elo = (r1 - emid.astype(jnp.float32)).astype(jnp.bfloat16)
    hi_ref[...] = ehi
    mid_ref[...] = emid
    lo_ref[...] = elo


def kernel(x, sample_rate, codebooks):
    B, C, T = x.shape
    esq = jnp.sum(codebooks ** 2, axis=-1)
    cbt = jnp.transpose(codebooks, (0, 2, 1))
    ehi_t, emid_t, elo_t = pl.pallas_call(
        _plane_kernel,
        out_shape=[jax.ShapeDtypeStruct((N_Q, DIM, BINS), jnp.bfloat16)] * 3,
    )(cbt)
    nt = pl.cdiv(T, TBLK)

    body = functools.partial(_rvq_kernel, t_total=T)

    quant, codes_bqT, loss_parts, sub_q = pl.pallas_call(
        body,
        grid=(B, nt),
        in_specs=[
            pl.BlockSpec((1, C, TBLK), lambda b, t: (b, 0, t)),
            pl.BlockSpec((N_Q, BINS, DIM), lambda b, t: (0, 0, 0)),
            pl.BlockSpec((N_Q, DIM, BINS), lambda b, t: (0, 0, 0)),
            pl.BlockSpec((N_Q, DIM, BINS), lambda b, t: (0, 0, 0)),
            pl.BlockSpec((N_Q, DIM, BINS), lambda b, t: (0, 0, 0)),
            pl.BlockSpec((N_Q, BINS), lambda b, t: (0, 0)),
        ],
        out_specs=[
            pl.BlockSpec((1, C, TBLK), lambda b, t: (b, 0, t)),
            pl.BlockSpec((1, N_Q, TBLK), lambda b, t: (b, 0, t)),
            pl.BlockSpec((1, N_Q, 128), lambda b, t: (b, 0, 0)),
            pl.BlockSpec((N_Q, 1, C, TBLK), lambda b, t: (0, b, 0, t)),
        ],
        out_shape=[
            jax.ShapeDtypeStruct((B, C, T), jnp.float32),
            jax.ShapeDtypeStruct((B, N_Q, T), jnp.int32),
            jax.ShapeDtypeStruct((B, N_Q, 128), jnp.float32),
            jax.ShapeDtypeStruct((N_Q, B, C, T), jnp.float32),
        ],
    )(x, codebooks, ehi_t, emid_t, elo_t, esq)

    codes_arr = jnp.transpose(codes_bqT, (1, 0, 2))
    commit_loss = jnp.sum(loss_parts[:, :, 0], axis=0) / (B * T * C)
    penalty = jnp.mean(commit_loss)
    sr = jnp.asarray(sample_rate, dtype=x.dtype)
    bw_per_q = np.log2(BINS).astype(np.float32) * sr / HOP
    bw = jnp.asarray(N_Q * bw_per_q, dtype=x.dtype)
    return quant, codes_arr, bw, penalty, sub_q


# speedup vs baseline: 1.0015x; 1.0015x over previous
import functools

import jax
import jax.numpy as jnp
import numpy as np
from jax.experimental import pallas as pl

DIM = 256
N_Q = 8
BINS = 1024
HOP = 320
TBLK = 512


def _rvq_kernel(x_ref, cb_ref, ehi_ref, emid_ref, elo_ref, esq_ref,
                quant_ref, codes_ref, loss_ref, subq_ref, *, t_total):
    t = pl.program_id(1)
    tb = x_ref.shape[2]
    hb = tb // 2
    iota16 = jax.lax.broadcasted_iota(jnp.int32, (BINS, hb), 0).astype(jnp.int16)
    lane = jax.lax.broadcasted_iota(jnp.int32, (1, hb), 1)
    dn = (((1,), (0,)), ((), ()))
    rs = [x_ref[0, :, pl.ds(h * hb, hb)] for h in range(2)]
    qsums = [jnp.zeros_like(rs[0]) for _ in range(2)]
    valids = [(t * tb + h * hb + lane) < t_total for h in range(2)]
    for q in range(N_Q):
        e = cb_ref[q]
        lrows = []
        for h in range(2):
            r = rs[h]
            eg = jax.lax.dot_general(
                e, r, dn, preferred_element_type=jnp.float32)
            rsq = jnp.sum(r * r, axis=0, keepdims=True)
            dist = (rsq - 2.0 * eg) + esq_ref[q][:, None]
            codes = jnp.argmin(dist, axis=0).astype(jnp.int32)
            onehot = jnp.where(iota16 == codes.astype(jnp.int16)[None, :],
                               jnp.bfloat16(1.0), jnp.bfloat16(0.0))
            v = ((jax.lax.dot_general(ehi_ref[q], onehot, dn,
                                      preferred_element_type=jnp.float32)
                  + jax.lax.dot_general(emid_ref[q], onehot, dn,
                                        preferred_element_type=jnp.float32))
                 + jax.lax.dot_general(elo_ref[q], onehot, dn,
                                       preferred_element_type=jnp.float32))
            d = v - r
            qsums[h] = qsums[h] + (r + d)
            dm = jnp.where(valids[h], d, 0.0)
            lrows.append(jnp.full((128,), jnp.sum(dm * dm), jnp.float32))
            codes_ref[0, q, pl.ds(h * hb, hb)] = codes
            subq_ref[q, 0, :, pl.ds(h * hb, hb)] = v
            rs[h] = r - v
        lrow = lrows[0] + lrows[1]
        loss_ref[0, q, :] = jnp.where(t == 0, lrow, loss_ref[0, q, :] + lrow)
    quant_ref[0, :, pl.ds(0, hb)] = qsums[0]
    quant_ref[0, :, pl.ds(hb, hb)] = qsums[1]


def _plane_kernel(cbt_ref, hi_ref, mid_ref, lo_ref):
    e = cbt_ref[...]
    ehi = e.astype(jnp.bfloat16)
    r1 = e - ehi.astype(jnp.float32)
    emid = r1.astype(jnp.bfloat16)
    elo = (r1 - emid.astype(jnp.float32)).astype(jnp.bfloat16)
    hi_ref[...] = ehi
    mid_ref[...] = emid
    lo_ref[...] = elo


def kernel(x, sample_rate, codebooks):
    B, C, T = x.shape
    esq = jnp.sum(codebooks ** 2, axis=-1)
    cbt = jnp.transpose(codebooks, (0, 2, 1))
    ehi_t, emid_t, elo_t = pl.pallas_call(
        _plane_kernel,
        out_shape=[jax.ShapeDtypeStruct((N_Q, DIM, BINS), jnp.bfloat16)] * 3,
    )(cbt)
    nt = pl.cdiv(T, TBLK)

    body = functools.partial(_rvq_kernel, t_total=T)

    quant, codes_bqT, loss_parts, sub_q = pl.pallas_call(
        body,
        grid=(B, nt),
        in_specs=[
            pl.BlockSpec((1, C, TBLK), lambda b, t: (b, 0, t)),
            pl.BlockSpec((N_Q, BINS, DIM), lambda b, t: (0, 0, 0)),
            pl.BlockSpec((N_Q, DIM, BINS), lambda b, t: (0, 0, 0)),
            pl.BlockSpec((N_Q, DIM, BINS), lambda b, t: (0, 0, 0)),
            pl.BlockSpec((N_Q, DIM, BINS), lambda b, t: (0, 0, 0)),
            pl.BlockSpec((N_Q, BINS), lambda b, t: (0, 0)),
        ],
        out_specs=[
            pl.BlockSpec((1, C, TBLK), lambda b, t: (b, 0, t)),
            pl.BlockSpec((1, N_Q, TBLK), lambda b, t: (b, 0, t)),
            pl.BlockSpec((1, N_Q, 128), lambda b, t: (b, 0, 0)),
            pl.BlockSpec((N_Q, 1, C, TBLK), lambda b, t: (0, b, 0, t)),
        ],
        out_shape=[
            jax.ShapeDtypeStruct((B, C, T), jnp.float32),
            jax.ShapeDtypeStruct((B, N_Q, T), jnp.int32),
            jax.ShapeDtypeStruct((B, N_Q, 128), jnp.float32),
            jax.ShapeDtypeStruct((N_Q, B, C, T), jnp.float32),
        ],
    )(x, codebooks, ehi_t, emid_t, elo_t, esq)

    codes_arr = jnp.transpose(codes_bqT, (1, 0, 2))
    commit_loss = jnp.sum(loss_parts[:, :, 0], axis=0) / (B * T * C)
    penalty = jnp.mean(commit_loss)
    sr = jnp.asarray(sample_rate, dtype=x.dtype)
    bw_per_q = np.log2(BINS).astype(np.float32) * sr / HOP
    bw = jnp.asarray(N_Q * bw_per_q, dtype=x.dtype)
    return quant, codes_arr, bw, penalty, sub_q


# speedup vs baseline: 1.0175x; 1.0159x over previous
import functools

import jax
import jax.numpy as jnp
import numpy as np
from jax.experimental import pallas as pl

DIM = 256
N_Q = 8
BINS = 1024
HOP = 320
TBLK = 512


def _rvq_kernel(x_ref, cb_ref, ehi_ref, emid_ref, elo_ref, esq_ref,
                quant_ref, codes_ref, loss_ref, subq_ref, *, t_total):
    t = pl.program_id(1)
    tb = x_ref.shape[2]
    hb = tb // 2
    iota16 = jax.lax.broadcasted_iota(jnp.int32, (BINS, hb), 0).astype(jnp.int16)
    lane = jax.lax.broadcasted_iota(jnp.int32, (1, hb), 1)
    dn = (((1,), (0,)), ((), ()))
    rs = [x_ref[0, :, pl.ds(h * hb, hb)] for h in range(2)]
    qsums = [jnp.zeros_like(rs[0]) for _ in range(2)]
    valids = [(t * tb + h * hb + lane) < t_total for h in range(2)]
    for q in range(N_Q):
        e = cb_ref[q]
        lrows = []
        for h in range(2):
            r = rs[h]
            eg = jax.lax.dot_general(
                e, r, dn, preferred_element_type=jnp.float32)
            rsq = jnp.sum(r * r, axis=0, keepdims=True)
            dist = (rsq - 2.0 * eg) + esq_ref[q][:, None]
            codes = jnp.argmin(dist, axis=0).astype(jnp.int32)
            onehot = jnp.where(iota16 == codes.astype(jnp.int16)[None, :],
                               jnp.bfloat16(1.0), jnp.bfloat16(0.0))
            v = ((jax.lax.dot_general(ehi_ref[q], onehot, dn,
                                      preferred_element_type=jnp.float32)
                  + jax.lax.dot_general(emid_ref[q], onehot, dn,
                                        preferred_element_type=jnp.float32))
                 + jax.lax.dot_general(elo_ref[q], onehot, dn,
                                       preferred_element_type=jnp.float32))
            d = v - r
            qsums[h] = qsums[h] + (r + d)
            dm = jnp.where(valids[h], d, 0.0)
            lrows.append(jnp.full((128,), jnp.sum(dm * dm), jnp.float32))
            codes_ref[0, q, pl.ds(h * hb, hb)] = codes
            subq_ref[q, 0, :, pl.ds(h * hb, hb)] = v
            rs[h] = r - v
        lrow = lrows[0] + lrows[1]
        loss_ref[0, q, :] = jnp.where(t == 0, lrow, loss_ref[0, q, :] + lrow)
    quant_ref[0, :, pl.ds(0, hb)] = qsums[0]
    quant_ref[0, :, pl.ds(hb, hb)] = qsums[1]


def _plane_kernel(cb_ref, hi_ref, mid_ref, lo_ref):
    for q in range(N_Q):
        e = jnp.transpose(cb_ref[q], (1, 0))
        ehi = e.astype(jnp.bfloat16)
        r1 = e - ehi.astype(jnp.float32)
        emid = r1.astype(jnp.bfloat16)
        elo = (r1 - emid.astype(jnp.float32)).astype(jnp.bfloat16)
        hi_ref[q] = ehi
        mid_ref[q] = emid
        lo_ref[q] = elo


def kernel(x, sample_rate, codebooks):
    B, C, T = x.shape
    esq = jnp.sum(codebooks ** 2, axis=-1)
    ehi_t, emid_t, elo_t = pl.pallas_call(
        _plane_kernel,
        out_shape=[jax.ShapeDtypeStruct((N_Q, DIM, BINS), jnp.bfloat16)] * 3,
    )(codebooks)
    nt = pl.cdiv(T, TBLK)

    body = functools.partial(_rvq_kernel, t_total=T)

    quant, codes_bqT, loss_parts, sub_q = pl.pallas_call(
        body,
        grid=(B, nt),
        in_specs=[
            pl.BlockSpec((1, C, TBLK), lambda b, t: (b, 0, t)),
            pl.BlockSpec((N_Q, BINS, DIM), lambda b, t: (0, 0, 0)),
            pl.BlockSpec((N_Q, DIM, BINS), lambda b, t: (0, 0, 0)),
            pl.BlockSpec((N_Q, DIM, BINS), lambda b, t: (0, 0, 0)),
            pl.BlockSpec((N_Q, DIM, BINS), lambda b, t: (0, 0, 0)),
            pl.BlockSpec((N_Q, BINS), lambda b, t: (0, 0)),
        ],
        out_specs=[
            pl.BlockSpec((1, C, TBLK), lambda b, t: (b, 0, t)),
            pl.BlockSpec((1, N_Q, TBLK), lambda b, t: (b, 0, t)),
            pl.BlockSpec((1, N_Q, 128), lambda b, t: (b, 0, 0)),
            pl.BlockSpec((N_Q, 1, C, TBLK), lambda b, t: (0, b, 0, t)),
        ],
        out_shape=[
            jax.ShapeDtypeStruct((B, C, T), jnp.float32),
            jax.ShapeDtypeStruct((B, N_Q, T), jnp.int32),
            jax.ShapeDtypeStruct((B, N_Q, 128), jnp.float32),
            jax.ShapeDtypeStruct((N_Q, B, C, T), jnp.float32),
        ],
    )(x, codebooks, ehi_t, emid_t, elo_t, esq)

    codes_arr = jnp.transpose(codes_bqT, (1, 0, 2))
    commit_loss = jnp.sum(loss_parts[:, :, 0], axis=0) / (B * T * C)
    penalty = jnp.mean(commit_loss)
    sr = jnp.asarray(sample_rate, dtype=x.dtype)
    bw_per_q = np.log2(BINS).astype(np.float32) * sr / HOP
    bw = jnp.asarray(N_Q * bw_per_q, dtype=x.dtype)
    return quant, codes_arr, bw, penalty, sub_q


# speedup vs baseline: 1.1618x; 1.1419x over previous
import functools

import jax
import jax.numpy as jnp
import numpy as np
from jax.experimental import pallas as pl

DIM = 256
N_Q = 8
BINS = 1024
HOP = 320
TBLK = 768


def _rvq_kernel(x_ref, cb_ref, ehi_ref, emid_ref, elo_ref, esq_ref,
                quant_ref, codes_ref, loss_ref, subq_ref, *, t_total):
    t = pl.program_id(1)
    tb = x_ref.shape[2]
    hb = tb // 3
    iota16 = jax.lax.broadcasted_iota(jnp.int32, (BINS, hb), 0).astype(jnp.int16)
    lane = jax.lax.broadcasted_iota(jnp.int32, (1, hb), 1)
    dn = (((1,), (0,)), ((), ()))
    rs = [x_ref[0, :, pl.ds(h * hb, hb)] for h in range(3)]
    qsums = [jnp.zeros_like(rs[0]) for _ in range(3)]
    valids = [(t * tb + h * hb + lane) < t_total for h in range(3)]
    for q in range(N_Q):
        e = cb_ref[q]
        lrows = []
        for h in range(3):
            r = rs[h]
            eg = jax.lax.dot_general(
                e, r, dn, preferred_element_type=jnp.float32)
            rsq = jnp.sum(r * r, axis=0, keepdims=True)
            dist = (rsq - 2.0 * eg) + esq_ref[q][:, None]
            codes = jnp.argmin(dist, axis=0).astype(jnp.int32)
            onehot = jnp.where(iota16 == codes.astype(jnp.int16)[None, :],
                               jnp.bfloat16(1.0), jnp.bfloat16(0.0))
            v = ((jax.lax.dot_general(ehi_ref[q], onehot, dn,
                                      preferred_element_type=jnp.float32)
                  + jax.lax.dot_general(emid_ref[q], onehot, dn,
                                        preferred_element_type=jnp.float32))
                 + jax.lax.dot_general(elo_ref[q], onehot, dn,
                                       preferred_element_type=jnp.float32))
            d = v - r
            qsums[h] = qsums[h] + (r + d)
            dm = jnp.where(valids[h], d, 0.0)
            lrows.append(jnp.full((128,), jnp.sum(dm * dm), jnp.float32))
            codes_ref[0, q, pl.ds(h * hb, hb)] = codes
            subq_ref[q, 0, :, pl.ds(h * hb, hb)] = v
            rs[h] = r - v
        lrow = (lrows[0] + lrows[1]) + lrows[2]
        loss_ref[0, q, :] = jnp.where(t == 0, lrow, loss_ref[0, q, :] + lrow)
    for h in range(3):
        quant_ref[0, :, pl.ds(h * hb, hb)] = qsums[h]


def _plane_kernel(cb_ref, hi_ref, mid_ref, lo_ref):
    for q in range(N_Q):
        e = jnp.transpose(cb_ref[q], (1, 0))
        ehi = e.astype(jnp.bfloat16)
        r1 = e - ehi.astype(jnp.float32)
        emid = r1.astype(jnp.bfloat16)
        elo = (r1 - emid.astype(jnp.float32)).astype(jnp.bfloat16)
        hi_ref[q] = ehi
        mid_ref[q] = emid
        lo_ref[q] = elo


def kernel(x, sample_rate, codebooks):
    B, C, T = x.shape
    esq = jnp.sum(codebooks ** 2, axis=-1)
    ehi_t, emid_t, elo_t = pl.pallas_call(
        _plane_kernel,
        out_shape=[jax.ShapeDtypeStruct((N_Q, DIM, BINS), jnp.bfloat16)] * 3,
    )(codebooks)
    nt = pl.cdiv(T, TBLK)

    body = functools.partial(_rvq_kernel, t_total=T)

    quant, codes_bqT, loss_parts, sub_q = pl.pallas_call(
        body,
        grid=(B, nt),
        in_specs=[
            pl.BlockSpec((1, C, TBLK), lambda b, t: (b, 0, t)),
            pl.BlockSpec((N_Q, BINS, DIM), lambda b, t: (0, 0, 0)),
            pl.BlockSpec((N_Q, DIM, BINS), lambda b, t: (0, 0, 0)),
            pl.BlockSpec((N_Q, DIM, BINS), lambda b, t: (0, 0, 0)),
            pl.BlockSpec((N_Q, DIM, BINS), lambda b, t: (0, 0, 0)),
            pl.BlockSpec((N_Q, BINS), lambda b, t: (0, 0)),
        ],
        out_specs=[
            pl.BlockSpec((1, C, TBLK), lambda b, t: (b, 0, t)),
            pl.BlockSpec((1, N_Q, TBLK), lambda b, t: (b, 0, t)),
            pl.BlockSpec((1, N_Q, 128), lambda b, t: (b, 0, 0)),
            pl.BlockSpec((N_Q, 1, C, TBLK), lambda b, t: (0, b, 0, t)),
        ],
        out_shape=[
            jax.ShapeDtypeStruct((B, C, T), jnp.float32),
            jax.ShapeDtypeStruct((B, N_Q, T), jnp.int32),
            jax.ShapeDtypeStruct((B, N_Q, 128), jnp.float32),
            jax.ShapeDtypeStruct((N_Q, B, C, T), jnp.float32),
        ],
    )(x, codebooks, ehi_t, emid_t, elo_t, esq)

    codes_arr = jnp.transpose(codes_bqT, (1, 0, 2))
    commit_loss = jnp.sum(loss_parts[:, :, 0], axis=0) / (B * T * C)
    penalty = jnp.mean(commit_loss)
    sr = jnp.asarray(sample_rate, dtype=x.dtype)
    bw_per_q = np.log2(BINS).astype(np.float32) * sr / HOP
    bw = jnp.asarray(N_Q * bw_per_q, dtype=x.dtype)
    return quant, codes_arr, bw, penalty, sub_q
